# baseline (speedup 1.0000x reference)
import jax
import jax.numpy as jnp
from jax import lax
from jax.experimental import pallas as pl
from jax.experimental.pallas import tpu as pltpu
from jax.experimental.pallas import tpu_sc as plsc

B, S, D = 4, 4096, 768
V_ROWS = 100
V_PAD = 128
LANES = 16
NC, NS = 2, 16
NW = NC * NS
SEQ_PER_W = S // NW
R = 32
NCHUNK = D // LANES


def _combined_body(lt_ref, var_ref, time_ref, out_ref):
    for b in range(B):
        row = time_ref[pl.ds(lt_ref[b], 1), :]
        out_ref[pl.ds(b * V_PAD, V_ROWS), :] = var_ref[...] + row


def _make_combined(lt_i32, var_table, time_table):
    grid_spec = pltpu.PrefetchScalarGridSpec(
        num_scalar_prefetch=1,
        grid=(1,),
        in_specs=[
            pl.BlockSpec((V_ROWS, D), lambda i, lt: (0, 0)),
            pl.BlockSpec((72, D), lambda i, lt: (0, 0)),
        ],
        out_specs=pl.BlockSpec((B * V_PAD, D), lambda i, lt: (0, 0)),
    )
    return pl.pallas_call(
        _combined_body,
        grid_spec=grid_spec,
        out_shape=jax.ShapeDtypeStruct((B * V_PAD, D), jnp.float32),
    )(lt_i32, var_table, time_table)


def _sc_body(x_hbm, idx_hbm, pos_hbm, comb_hbm, out_hbm,
             idx_all, acc0, acc1, gat0, gat1, pos_v,
             sem_g0, sem_g1, sem_x0, sem_x1, sem_s0, sem_s1, sem_p):
    wid = lax.axis_index("s") * NC + lax.axis_index("c")
    w0 = pl.multiple_of(wid * SEQ_PER_W, SEQ_PER_W)
    n_blk = (SEQ_PER_W // R) * B

    acc_b = (acc0, acc1)
    gat_b = (gat0, gat1)
    sem_g = (sem_g0, sem_g1)
    sem_x = (sem_x0, sem_x1)
    sem_s = (sem_s0, sem_s1)

    pltpu.sync_copy(idx_hbm.at[:, pl.ds(w0, SEQ_PER_W)], idx_all)
    for b in range(1, B):
        for jj in range(SEQ_PER_W // LANES):
            sl = pl.ds(jj * LANES, LANES)
            idx_all[b, sl] = idx_all[b, sl] + b * V_PAD

    def s0_of(t):
        return pl.multiple_of(w0 + (t // B) * R, R)

    def idx_view(t):
        return idx_all.at[t % B, pl.ds(pl.multiple_of((t // B) * R, R), R)]

    def issue_loads(t, k):
        pltpu.async_copy(comb_hbm.at[idx_view(t)], gat_b[k], sem_g[k])
        pltpu.make_async_copy(x_hbm.at[0, pl.ds(0, R), :],
                              acc_b[k], sem_s[k]).wait()
        pltpu.async_copy(x_hbm.at[t % B, pl.ds(s0_of(t), R), :],
                         acc_b[k], sem_x[k])

    def drain(sem, buf):
        pltpu.make_async_copy(x_hbm.at[0, pl.ds(0, R), :], buf, sem).wait()

    def compute_and_store(t, k):
        drain(sem_g[k], gat_b[k])
        drain(sem_x[k], acc_b[k])
        acc, gat = acc_b[k], gat_b[k]

        def row(r, c2):
            for j in range(NCHUNK):
                sl = pl.ds(j * LANES, LANES)
                acc[r, sl] = acc[r, sl] + gat[r, sl] + pos_v[r, sl]
            return c2

        lax.fori_loop(0, R, row, 0)
        pltpu.async_copy(acc, out_hbm.at[t % B, pl.ds(s0_of(t), R), :],
                         sem_s[k])

    pltpu.sync_copy(pos_hbm.at[pl.ds(s0_of(0), R), :], pos_v)
    pltpu.async_copy(comb_hbm.at[idx_view(0)], gat_b[0], sem_g[0])
    pltpu.async_copy(x_hbm.at[0, pl.ds(s0_of(0), R), :], acc_b[0], sem_x[0])

    def pair(g, carry):
        t0 = g * 2
        t1 = t0 + 1

        pltpu.async_copy(comb_hbm.at[idx_view(t1)], gat_b[1], sem_g[1])

        @pl.when(g >= 1)
        def _():
            drain(sem_s[1], acc_b[1])
        pltpu.async_copy(x_hbm.at[t1 % B, pl.ds(s0_of(t1), R), :],
                         acc_b[1], sem_x[1])

        @pl.when(jnp.logical_and(g > 0, g % 2 == 0))
        def _():
            drain(sem_p, pos_v)
        compute_and_store(t0, 0)

        @pl.when(g < (n_blk // 2 - 1))
        def _():
            issue_loads(t0 + 2, 0)
        drain(sem_g[1], gat_b[1])
        drain(sem_x[1], acc_b[1])
        acc, gat = acc_b[1], gat_b[1]

        def row(r, c2):
            for j in range(NCHUNK):
                sl = pl.ds(j * LANES, LANES)
                acc[r, sl] = acc[r, sl] + gat[r, sl] + pos_v[r, sl]
            return c2

        lax.fori_loop(0, R, row, 0)
        pltpu.async_copy(acc, out_hbm.at[t1 % B, pl.ds(s0_of(t1), R), :],
                         sem_s[1])

        @pl.when(jnp.logical_and(g % 2 == 1, g < n_blk // 2 - 1))
        def _():
            pltpu.async_copy(pos_hbm.at[pl.ds(s0_of(t1 + 1), R), :],
                             pos_v, sem_p)
        return carry

    lax.fori_loop(0, n_blk // 2, pair, 0)

    drain(sem_s[0], acc_b[0])
    drain(sem_s[1], acc_b[1])


_sc_call = pl.kernel(
    _sc_body,
    out_type=jax.ShapeDtypeStruct((B, S, D), jnp.float32),
    mesh=plsc.VectorSubcoreMesh(core_axis_name="c", subcore_axis_name="s"),
    scratch_types=[
        pltpu.VMEM((B, SEQ_PER_W), jnp.int32),
        pltpu.VMEM((R, D), jnp.float32),
        pltpu.VMEM((R, D), jnp.float32),
        pltpu.VMEM((R, D), jnp.float32),
        pltpu.VMEM((R, D), jnp.float32),
        pltpu.VMEM((R, D), jnp.float32),
        pltpu.SemaphoreType.DMA,
        pltpu.SemaphoreType.DMA,
        pltpu.SemaphoreType.DMA,
        pltpu.SemaphoreType.DMA,
        pltpu.SemaphoreType.DMA,
        pltpu.SemaphoreType.DMA,
        pltpu.SemaphoreType.DMA,
    ],
)


def kernel(x, variable, pos_emb, lead_time, var_table, time_table):
    variable = variable.astype(jnp.int32)
    lt = lead_time.reshape(-1).astype(jnp.int32)
    combined = _make_combined(lt, var_table, time_table)
    pos2d = pos_emb.reshape(S, D)
    return _sc_call(x, variable, pos2d, combined)


# speedup vs baseline: 1.2142x; 1.2142x over previous
import functools

import jax
import jax.numpy as jnp
from jax import lax
from jax.experimental import pallas as pl
from jax.experimental.pallas import tpu as pltpu
from jax.experimental.pallas import tpu_sc as plsc

B, S, D = 4, 4096, 768
V_ROWS = 100
V_PAD = 128
LANES = 16
NC, NS = 2, 16
NW = NC * NS
SEQ_PER_W = S // NW
R = 32
NCHUNK = D // LANES


def _combined_body(lt_ref, var_ref, time_ref, out_ref):
    for b in range(B):
        row = time_ref[pl.ds(lt_ref[b], 1), :]
        out_ref[pl.ds(b * V_PAD, V_ROWS), :] = var_ref[...] + row


def _make_combined(lt_i32, var_table, time_table):
    grid_spec = pltpu.PrefetchScalarGridSpec(
        num_scalar_prefetch=1,
        grid=(1,),
        in_specs=[
            pl.BlockSpec((V_ROWS, D), lambda i, lt: (0, 0)),
            pl.BlockSpec((72, D), lambda i, lt: (0, 0)),
        ],
        out_specs=pl.BlockSpec((B * V_PAD, D), lambda i, lt: (0, 0)),
    )
    return pl.pallas_call(
        _combined_body,
        grid_spec=grid_spec,
        out_shape=jax.ShapeDtypeStruct((B * V_PAD, D), jnp.float32),
    )(lt_i32, var_table, time_table)


def _sc_body(x_hbm, idx_hbm, pos_hbm, comb_hbm, out_hbm,
             idx0, idx1, acc0, acc1, gat0, gat1, pos_v,
             sem_g0, sem_g1, sem_x0, sem_x1, sem_s0, sem_s1, sem_p):
    wid = lax.axis_index("s") * NC + lax.axis_index("c")
    w0 = wid * SEQ_PER_W

    n_blk = (SEQ_PER_W // R) * B

    idx_b = (idx0, idx1)
    acc_b = (acc0, acc1)
    gat_b = (gat0, gat1)
    sem_g = (sem_g0, sem_g1)
    sem_x = (sem_x0, sem_x1)
    sem_s = (sem_s0, sem_s1)

    def s0_of(t):
        return pl.multiple_of(w0 + (t // B) * R, R)

    def load_idx(t, k):
        b = t % B
        ref = idx_b[k]
        pltpu.sync_copy(idx_hbm.at[b, pl.ds(s0_of(t), R)], ref)
        for jj in range(R // LANES):
            sl = pl.ds(jj * LANES, LANES)
            ref[sl] = ref[sl] + b * V_PAD

    def issue_loads(t, k):
        load_idx(t, k)
        pltpu.async_copy(comb_hbm.at[idx_b[k]], gat_b[k], sem_g[k])
        pltpu.make_async_copy(x_hbm.at[0, pl.ds(0, R), :],
                              acc_b[k], sem_s[k]).wait()
        pltpu.async_copy(x_hbm.at[t % B, pl.ds(s0_of(t), R), :],
                         acc_b[k], sem_x[k])

    def drain(sem, buf):
        pltpu.make_async_copy(x_hbm.at[0, pl.ds(0, R), :], buf, sem).wait()

    def compute_and_store(t, k):
        drain(sem_g[k], gat_b[k])
        drain(sem_x[k], acc_b[k])
        acc, gat = acc_b[k], gat_b[k]

        def row(r, c2):
            for j in range(NCHUNK):
                sl = pl.ds(j * LANES, LANES)
                plsc.addupdate(acc.at[r, sl], gat[r, sl] + pos_v[r, sl])
            return c2

        lax.fori_loop(0, R, row, 0)
        pltpu.async_copy(acc, out_hbm.at[t % B, pl.ds(s0_of(t), R), :],
                         sem_s[k])

    pltpu.sync_copy(pos_hbm.at[pl.ds(s0_of(0), R), :], pos_v)
    load_idx(0, 0)
    pltpu.async_copy(comb_hbm.at[idx_b[0]], gat_b[0], sem_g[0])
    pltpu.async_copy(x_hbm.at[0, pl.ds(s0_of(0), R), :], acc_b[0], sem_x[0])

    def pair(g, carry):
        t0 = g * 2
        t1 = t0 + 1

        load_idx(t1, 1)
        pltpu.async_copy(comb_hbm.at[idx_b[1]], gat_b[1], sem_g[1])

        @pl.when(g >= 1)
        def _():
            drain(sem_s[1], acc_b[1])
        pltpu.async_copy(x_hbm.at[t1 % B, pl.ds(s0_of(t1), R), :],
                         acc_b[1], sem_x[1])

        @pl.when(jnp.logical_and(g > 0, g % 2 == 0))
        def _():
            drain(sem_p, pos_v)
        compute_and_store(t0, 0)

        @pl.when(g < (n_blk // 2 - 1))
        def _():
            issue_loads(t0 + 2, 0)
        drain(sem_g[1], gat_b[1])
        drain(sem_x[1], acc_b[1])
        acc, gat = acc_b[1], gat_b[1]

        def row(r, c2):
            for j in range(NCHUNK):
                sl = pl.ds(j * LANES, LANES)
                plsc.addupdate(acc.at[r, sl], gat[r, sl] + pos_v[r, sl])
            return c2

        lax.fori_loop(0, R, row, 0)
        pltpu.async_copy(acc, out_hbm.at[t1 % B, pl.ds(s0_of(t1), R), :],
                         sem_s[1])

        @pl.when(jnp.logical_and(g % 2 == 1, g < n_blk // 2 - 1))
        def _():
            pltpu.async_copy(pos_hbm.at[pl.ds(s0_of(t1 + 1), R), :],
                             pos_v, sem_p)
        return carry

    lax.fori_loop(0, n_blk // 2, pair, 0)

    drain(sem_s[0], acc_b[0])
    drain(sem_s[1], acc_b[1])


_sc_call = pl.kernel(
    _sc_body,
    out_type=jax.ShapeDtypeStruct((B, S, D), jnp.float32),
    mesh=plsc.VectorSubcoreMesh(core_axis_name="c", subcore_axis_name="s"),
    scratch_types=[
        pltpu.VMEM((R,), jnp.int32),
        pltpu.VMEM((R,), jnp.int32),
        pltpu.VMEM((R, D), jnp.float32),
        pltpu.VMEM((R, D), jnp.float32),
        pltpu.VMEM((R, D), jnp.float32),
        pltpu.VMEM((R, D), jnp.float32),
        pltpu.VMEM((R, D), jnp.float32),
        pltpu.SemaphoreType.DMA,
        pltpu.SemaphoreType.DMA,
        pltpu.SemaphoreType.DMA,
        pltpu.SemaphoreType.DMA,
        pltpu.SemaphoreType.DMA,
        pltpu.SemaphoreType.DMA,
        pltpu.SemaphoreType.DMA,
    ],
)


def kernel(x, variable, pos_emb, lead_time, var_table, time_table):
    variable = variable.astype(jnp.int32)
    lt = lead_time.reshape(-1).astype(jnp.int32)
    combined = _make_combined(lt, var_table, time_table)
    pos2d = pos_emb.reshape(S, D)
    return _sc_call(x, variable, pos2d, combined)
